# baseline (speedup 1.0000x reference)
import numpy as np
import jax
import jax.numpy as jnp
from jax import lax
from jax.experimental import pallas as pl
from jax.experimental.pallas import tpu as pltpu

B, P, D, H, L = 1024, 22, 3, 64, 4
Pp = 24
Q = 8
W2 = 2 * H
NR = Q * Pp
R = Q * Pp * Pp

_PAT26 = np.kron(np.eye(2), np.ones((1, 3))).astype(np.float32)
_PAT62 = np.kron(np.eye(2), np.ones((3, 1))).astype(np.float32)
_PAT2H = np.kron(np.eye(2), np.ones((1, H))).astype(np.float32)
_EXACT = jax.lax.Precision.HIGHEST


def _sh(t):
    return t * (jnp.tanh(t) + 1.0)


def _egnn_block(t_ref, x_ref, hi_ref, embWh_ref, embWt_ref, embb_ref,
                p26_ref, p62_ref, p2h_ref,
                eW1r_ref, eW1c_ref, Wr2_ref, We2_ref, eb1_ref,
                eW2_ref, eb2_ref, cW1_ref, cb1_ref, cW2_ref,
                nW1h_ref, nW1a_ref, nb1_ref, nW2_ref, nb2_ref, o_ref):
    f32 = jnp.float32
    x2 = x_ref[:]

    base = jnp.dot(hi_ref[:], embWh_ref[:], preferred_element_type=f32)
    base = base + embb_ref[:]
    baseD = jnp.concatenate([base, base], axis=-1)
    t64 = jnp.dot(t_ref[:], p2h_ref[:], preferred_element_type=f32)
    wtD = jnp.concatenate([embWt_ref[:], embWt_ref[:]], axis=-1)
    h2 = (lax.broadcast_in_dim(baseD, (Q, Pp, W2), (1, 2))
          + lax.broadcast_in_dim(t64, (Q, Pp, W2), (0, 2))
          * wtD.reshape(1, 1, W2))
    hn = h2.reshape(NR, W2)

    nvalid = (lax.broadcasted_iota(jnp.int32, (1, Pp, 1), 1) < P)
    nmask3 = nvalid.astype(f32)
    nkill = (1.0 - nmask3) * (-1e30)

    def cdiff(c2):
        ci = lax.broadcast_in_dim(c2, (Q, Pp, Pp, 2 * D), (0, 1, 3))
        cj = lax.broadcast_in_dim(c2, (Q, Pp, Pp, 2 * D), (0, 2, 3))
        return ci - cj

    def radial2(cd):
        sq = (cd * cd).reshape(R, 2 * D)
        return jnp.dot(sq, p62_ref[:], preferred_element_type=f32,
                       precision=_EXACT)

    cd0 = cdiff(x2)
    ea2 = radial2(cd0)

    coord2 = x2
    for l in range(L):
        if l == 0:
            cd, rad2 = cd0, ea2
        else:
            cd = cdiff(coord2)
            rad2 = radial2(cd)

        A = (jnp.dot(hn, eW1r_ref[l], preferred_element_type=f32)
             + eb1_ref[l])
        C = jnp.dot(hn, eW1c_ref[l], preferred_element_type=f32)
        me = _sh(eb2_ref[l])
        cstar = jnp.dot(_sh(jnp.dot(me, cW1_ref[l],
                                    preferred_element_type=f32)
                            + cb1_ref[l]),
                        cW2_ref[l], preferred_element_type=f32)
        m1d = _sh(A + C)
        md = _sh(jnp.dot(m1d, eW2_ref[l], preferred_element_type=f32)
                 + eb2_ref[l])
        A3 = A.reshape(Q, Pp, W2)
        C3 = C.reshape(Q, Pp, W2) + nkill
        Ab = lax.broadcast_in_dim(A3, (Q, Pp, Pp, W2), (0, 1, 3)).reshape(R, W2)
        Cb = lax.broadcast_in_dim(C3, (Q, Pp, Pp, W2), (0, 2, 3)).reshape(R, W2)
        z1 = (Ab + Cb
              + jnp.dot(rad2, Wr2_ref[l], preferred_element_type=f32)
              + jnp.dot(ea2, We2_ref[l], preferred_element_type=f32))
        m1 = _sh(z1)
        m = _sh(jnp.dot(m1, eW2_ref[l], preferred_element_type=f32)
                + eb2_ref[l])
        c1 = _sh(jnp.dot(m, cW1_ref[l], preferred_element_type=f32)
                 + cb1_ref[l])
        cmp = jnp.dot(c1, cW2_ref[l], preferred_element_type=f32)
        cm6 = jnp.dot(cmp, p26_ref[:], preferred_element_type=f32,
                      precision=_EXACT)
        upd = jnp.sum(cd * cm6.reshape(Q, Pp, Pp, 2 * D), axis=2)
        cs6 = jnp.dot(cstar, p26_ref[:], preferred_element_type=f32,
                      precision=_EXACT).reshape(1, 1, 2 * D)
        coord2 = (coord2 + upd - 2.0 * cs6 * coord2) * nmask3

        if l < L - 1:
            agg = (jnp.sum(m.reshape(Q, Pp, Pp, W2), axis=2)
                   .reshape(NR, W2)) - md - 2.0 * me
            zn = (jnp.dot(hn, nW1h_ref[l], preferred_element_type=f32)
                  + jnp.dot(agg, nW1a_ref[l], preferred_element_type=f32)
                  + nb1_ref[l])
            hn = hn + jnp.dot(_sh(zn), nW2_ref[l],
                              preferred_element_type=f32) + nb2_ref[l]

    vel2 = coord2 - x2
    mean = jnp.sum(vel2 * nmask3, axis=1, keepdims=True) * (1.0 / P)
    o_ref[:] = vel2 - mean


def _blkdiag(Wl):
    z = jnp.zeros_like(Wl)
    top = jnp.concatenate([Wl, z], axis=-1)
    bot = jnp.concatenate([z, Wl], axis=-1)
    return jnp.concatenate([top, bot], axis=1)


def _dup(bl):
    return jnp.concatenate([bl, bl], axis=-1)


def kernel(t, xs, h_initial, edges, emb_W, emb_b, out_W, out_b,
           eW1, eb1, eW2, eb2, nW1, nb1, nW2, nb2, cW1, cb1, cW2):
    del edges, out_W, out_b
    f32 = jnp.float32
    half = B // 2
    x = xs.reshape(B, P, D)
    xpad = jnp.pad(x, ((0, 0), (0, Pp - P), (0, 0)))
    xpack = (xpad.reshape(half, 2, Pp, D).transpose(0, 2, 1, 3)
             .reshape(half, Pp, 2 * D))
    tpack = t.reshape(half, 2)
    hi_pad = jnp.pad(h_initial, ((0, Pp - P), (0, 0)))

    embWh = emb_W[:4]
    embWt = emb_W[4:5].astype(jnp.bfloat16).astype(f32)
    embb = emb_b.reshape(1, H)
    eW1r = _blkdiag(0.5 * eW1[:, :H, :])
    eW1c = _blkdiag(0.5 * eW1[:, H:2 * H, :])
    Wr2 = _blkdiag(0.5 * eW1[:, 2 * H:2 * H + 1, :])
    We2 = _blkdiag(0.5 * eW1[:, 2 * H + 1:, :])
    eb1r = _dup(0.5 * eb1.reshape(L, 1, H))
    eW2d = _blkdiag(0.5 * eW2)
    eb2d = _dup(0.5 * eb2.reshape(L, 1, H))
    cW1d = _blkdiag(0.5 * cW1)
    cb1d = _dup(0.5 * cb1.reshape(L, 1, H))
    cW2d = _blkdiag(cW2)
    nW1h = _blkdiag(0.5 * nW1[:, :H, :])
    nW1a = _blkdiag(0.5 * nW1[:, H:, :])
    nb1r = _dup(0.5 * nb1.reshape(L, 1, H))
    nW2d = _blkdiag(nW2)
    nb2r = _dup(nb2.reshape(L, 1, H))
    p26 = jnp.asarray(_PAT26)
    p62 = jnp.asarray(_PAT62)
    p2h = jnp.asarray(_PAT2H)

    G = half // Q
    full = lambda *shape: pl.BlockSpec(shape, lambda g: (0,) * len(shape))
    out = pl.pallas_call(
        _egnn_block,
        grid=(G,),
        in_specs=[
            pl.BlockSpec((Q, 2), lambda g: (g, 0)),
            pl.BlockSpec((Q, Pp, 2 * D), lambda g: (g, 0, 0)),
            full(Pp, 4),
            full(4, H), full(1, H), full(1, H),
            full(2, 2 * D), full(2 * D, 2), full(2, W2),
            full(L, W2, W2), full(L, W2, W2),
            full(L, 2, W2), full(L, 2, W2), full(L, 1, W2),
            full(L, W2, W2), full(L, 1, W2),
            full(L, W2, W2), full(L, 1, W2), full(L, W2, 2),
            full(L, W2, W2), full(L, W2, W2), full(L, 1, W2),
            full(L, W2, W2), full(L, 1, W2),
        ],
        out_specs=pl.BlockSpec((Q, Pp, 2 * D), lambda g: (g, 0, 0)),
        out_shape=jax.ShapeDtypeStruct((half, Pp, 2 * D), f32),
        compiler_params=pltpu.CompilerParams(
            dimension_semantics=("parallel",)),
    )(tpack, xpack, hi_pad, embWh, embWt, embb, p26, p62, p2h,
      eW1r, eW1c, Wr2, We2, eb1r, eW2d, eb2d, cW1d, cb1d, cW2d,
      nW1h, nW1a, nb1r, nW2d, nb2r)
    vel = (out.reshape(half, Pp, 2, D).transpose(0, 2, 1, 3)
           .reshape(B, Pp, D))
    return vel[:, :P, :].reshape(B, P * D)


# speedup vs baseline: 2.7679x; 2.7679x over previous
import numpy as np
import jax
import jax.numpy as jnp
from jax import lax
from jax.experimental import pallas as pl
from jax.experimental.pallas import tpu as pltpu

B, P, D, H, L = 1024, 22, 3, 64, 4
Pp = 24
Q = 8
W2 = 2 * H
NR = Q * Pp
R = Q * Pp * Pp

_PAT62 = np.kron(np.eye(2), np.ones((3, 1))).astype(np.float32)
_PAT2H = np.kron(np.eye(2), np.ones((1, H))).astype(np.float32)


def _sh(t):
    return t * (jnp.tanh(t) + 1.0)


def _egnn_block(t_ref, x_ref, hi_ref, embWh_ref, embWt_ref, embb_ref,
                p62_ref, p2h_ref,
                eW1r_ref, eW1c_ref, Wr2_ref, We2_ref, eb1_ref,
                eW2_ref, eb2_ref, cW1_ref, cb1_ref, cW26_ref,
                nW1h_ref, nW1a_ref, nb1_ref, nW2_ref, nb2_ref, o_ref):
    f32 = jnp.float32
    x2 = x_ref[:]

    base = jnp.dot(hi_ref[:], embWh_ref[:], preferred_element_type=f32)
    base = base + embb_ref[:]
    baseD = jnp.concatenate([base, base], axis=-1)
    t64 = jnp.dot(t_ref[:], p2h_ref[:], preferred_element_type=f32)
    wtD = jnp.concatenate([embWt_ref[:], embWt_ref[:]], axis=-1)
    h2 = (lax.broadcast_in_dim(baseD, (Q, Pp, W2), (1, 2))
          + lax.broadcast_in_dim(t64, (Q, Pp, W2), (0, 2))
          * wtD.reshape(1, 1, W2))
    hn = h2.reshape(NR, W2)

    nvalid = (lax.broadcasted_iota(jnp.int32, (1, Pp, 1), 1) < P)
    nmask3 = nvalid.astype(f32)
    nkill = (1.0 - nmask3) * (-1e30)

    def cdiff(c2):
        ci = lax.broadcast_in_dim(c2, (Q, Pp, Pp, 2 * D), (0, 1, 3))
        cj = lax.broadcast_in_dim(c2, (Q, Pp, Pp, 2 * D), (0, 2, 3))
        return ci - cj

    def radial2(cd):
        sq = (cd * cd).reshape(R, 2 * D)
        err = sq - sq.astype(jnp.bfloat16).astype(f32)
        return (jnp.dot(sq, p62_ref[:], preferred_element_type=f32)
                + jnp.dot(err, p62_ref[:], preferred_element_type=f32))

    cd0 = cdiff(x2)
    ea2 = radial2(cd0)

    coord2 = x2
    for l in range(L):
        if l == 0:
            cd, rad2 = cd0, ea2
        else:
            cd = cdiff(coord2)
            rad2 = radial2(cd)

        A = (jnp.dot(hn, eW1r_ref[l], preferred_element_type=f32)
             + eb1_ref[l])
        C = jnp.dot(hn, eW1c_ref[l], preferred_element_type=f32)
        me = _sh(eb2_ref[l])
        cs6 = jnp.dot(_sh(jnp.dot(me, cW1_ref[l],
                                  preferred_element_type=f32)
                          + cb1_ref[l]),
                      cW26_ref[l], preferred_element_type=f32)
        m1d = _sh(A + C)
        md = _sh(jnp.dot(m1d, eW2_ref[l], preferred_element_type=f32)
                 + eb2_ref[l])
        A3 = A.reshape(Q, Pp, W2)
        C3 = C.reshape(Q, Pp, W2) + nkill
        Ab = lax.broadcast_in_dim(A3, (Q, Pp, Pp, W2), (0, 1, 3)).reshape(R, W2)
        Cb = lax.broadcast_in_dim(C3, (Q, Pp, Pp, W2), (0, 2, 3)).reshape(R, W2)
        z1 = (Ab + Cb
              + jnp.dot(rad2, Wr2_ref[l], preferred_element_type=f32)
              + jnp.dot(ea2, We2_ref[l], preferred_element_type=f32))
        m1 = _sh(z1)
        m = _sh(jnp.dot(m1, eW2_ref[l], preferred_element_type=f32)
                + eb2_ref[l])
        c1 = _sh(jnp.dot(m, cW1_ref[l], preferred_element_type=f32)
                 + cb1_ref[l])
        cm6 = jnp.dot(c1, cW26_ref[l], preferred_element_type=f32)
        upd = jnp.sum(cd * cm6.reshape(Q, Pp, Pp, 2 * D), axis=2)
        coord2 = (coord2 + upd
                  - 2.0 * cs6.reshape(1, 1, 2 * D) * coord2) * nmask3

        if l < L - 1:
            agg = (jnp.sum(m.reshape(Q, Pp, Pp, W2), axis=2)
                   .reshape(NR, W2)) - md - 2.0 * me
            zn = (jnp.dot(hn, nW1h_ref[l], preferred_element_type=f32)
                  + jnp.dot(agg, nW1a_ref[l], preferred_element_type=f32)
                  + nb1_ref[l])
            hn = hn + jnp.dot(_sh(zn), nW2_ref[l],
                              preferred_element_type=f32) + nb2_ref[l]

    vel2 = coord2 - x2
    mean = jnp.sum(vel2 * nmask3, axis=1, keepdims=True) * (1.0 / P)
    o_ref[:] = vel2 - mean


def _blkdiag(Wl):
    z = jnp.zeros_like(Wl)
    top = jnp.concatenate([Wl, z], axis=-1)
    bot = jnp.concatenate([z, Wl], axis=-1)
    return jnp.concatenate([top, bot], axis=1)


def _dup(bl):
    return jnp.concatenate([bl, bl], axis=-1)


def kernel(t, xs, h_initial, edges, emb_W, emb_b, out_W, out_b,
           eW1, eb1, eW2, eb2, nW1, nb1, nW2, nb2, cW1, cb1, cW2):
    del edges, out_W, out_b
    f32 = jnp.float32
    half = B // 2
    x = xs.reshape(B, P, D)
    xpad = jnp.pad(x, ((0, 0), (0, Pp - P), (0, 0)))
    xpack = (xpad.reshape(half, 2, Pp, D).transpose(0, 2, 1, 3)
             .reshape(half, Pp, 2 * D))
    tpack = t.reshape(half, 2)
    hi_pad = jnp.pad(h_initial, ((0, Pp - P), (0, 0)))

    embWh = emb_W[:4]
    embWt = emb_W[4:5].astype(jnp.bfloat16).astype(f32)
    embb = emb_b.reshape(1, H)
    eW1r = _blkdiag(0.5 * eW1[:, :H, :])
    eW1c = _blkdiag(0.5 * eW1[:, H:2 * H, :])
    Wr2 = _blkdiag(0.5 * eW1[:, 2 * H:2 * H + 1, :])
    We2 = _blkdiag(0.5 * eW1[:, 2 * H + 1:, :])
    eb1r = _dup(0.5 * eb1.reshape(L, 1, H))
    eW2d = _blkdiag(0.5 * eW2)
    eb2d = _dup(0.5 * eb2.reshape(L, 1, H))
    cW1d = _blkdiag(0.5 * cW1)
    cb1d = _dup(0.5 * cb1.reshape(L, 1, H))
    cw2rep = jnp.tile(cW2, (1, 1, 3))
    zw = jnp.zeros_like(cw2rep)
    cW26d = jnp.concatenate([jnp.concatenate([cw2rep, zw], -1),
                             jnp.concatenate([zw, cw2rep], -1)], 1)
    nW1h = _blkdiag(0.5 * nW1[:, :H, :])
    nW1a = _blkdiag(0.5 * nW1[:, H:, :])
    nb1r = _dup(0.5 * nb1.reshape(L, 1, H))
    nW2d = _blkdiag(nW2)
    nb2r = _dup(nb2.reshape(L, 1, H))
    p62 = jnp.asarray(_PAT62)
    p2h = jnp.asarray(_PAT2H)

    G = half // Q
    full = lambda *shape: pl.BlockSpec(shape, lambda g: (0,) * len(shape))
    out = pl.pallas_call(
        _egnn_block,
        grid=(G,),
        in_specs=[
            pl.BlockSpec((Q, 2), lambda g: (g, 0)),
            pl.BlockSpec((Q, Pp, 2 * D), lambda g: (g, 0, 0)),
            full(Pp, 4),
            full(4, H), full(1, H), full(1, H),
            full(2 * D, 2), full(2, W2),
            full(L, W2, W2), full(L, W2, W2),
            full(L, 2, W2), full(L, 2, W2), full(L, 1, W2),
            full(L, W2, W2), full(L, 1, W2),
            full(L, W2, W2), full(L, 1, W2), full(L, W2, 2 * D),
            full(L, W2, W2), full(L, W2, W2), full(L, 1, W2),
            full(L, W2, W2), full(L, 1, W2),
        ],
        out_specs=pl.BlockSpec((Q, Pp, 2 * D), lambda g: (g, 0, 0)),
        out_shape=jax.ShapeDtypeStruct((half, Pp, 2 * D), f32),
        compiler_params=pltpu.CompilerParams(
            dimension_semantics=("parallel",)),
    )(tpack, xpack, hi_pad, embWh, embWt, embb, p62, p2h,
      eW1r, eW1c, Wr2, We2, eb1r, eW2d, eb2d, cW1d, cb1d, cW26d,
      nW1h, nW1a, nb1r, nW2d, nb2r)
    vel = (out.reshape(half, Pp, 2, D).transpose(0, 2, 1, 3)
           .reshape(B, Pp, D))
    return vel[:, :P, :].reshape(B, P * D)


# speedup vs baseline: 2.9065x; 1.0501x over previous
import numpy as np
import jax
import jax.numpy as jnp
from jax import lax
from jax.experimental import pallas as pl
from jax.experimental.pallas import tpu as pltpu

B, P, D, H, L = 1024, 22, 3, 64, 4
Pp = 24
Q = 16
W2 = 2 * H
NR = Q * Pp
R = Q * Pp * Pp

_PAT62 = np.kron(np.eye(2), np.ones((3, 1))).astype(np.float32)
_PAT2H = np.kron(np.eye(2), np.ones((1, H))).astype(np.float32)


def _sh(t):
    return t * (jnp.tanh(t) + 1.0)


def _egnn_block(t_ref, x_ref, hi_ref, embWh_ref, embWt_ref, embb_ref,
                p62_ref, p2h_ref,
                eW1r_ref, eW1c_ref, Wr2_ref, We2_ref, eb1_ref,
                eW2_ref, eb2_ref, cW1_ref, cb1_ref, cW26_ref,
                nW1h_ref, nW1a_ref, nb1_ref, nW2_ref, nb2_ref, o_ref):
    f32 = jnp.float32
    x2 = x_ref[:]

    base = jnp.dot(hi_ref[:], embWh_ref[:], preferred_element_type=f32)
    base = base + embb_ref[:]
    baseD = jnp.concatenate([base, base], axis=-1)
    t64 = jnp.dot(t_ref[:], p2h_ref[:], preferred_element_type=f32)
    wtD = jnp.concatenate([embWt_ref[:], embWt_ref[:]], axis=-1)
    h2 = (lax.broadcast_in_dim(baseD, (Q, Pp, W2), (1, 2))
          + lax.broadcast_in_dim(t64, (Q, Pp, W2), (0, 2))
          * wtD.reshape(1, 1, W2))
    hn = h2.reshape(NR, W2)

    nvalid = (lax.broadcasted_iota(jnp.int32, (1, Pp, 1), 1) < P)
    nmask3 = nvalid.astype(f32)
    nkill = (1.0 - nmask3) * (-1e30)

    def cdiff(c2):
        ci = lax.broadcast_in_dim(c2, (Q, Pp, Pp, 2 * D), (0, 1, 3))
        cj = lax.broadcast_in_dim(c2, (Q, Pp, Pp, 2 * D), (0, 2, 3))
        return ci - cj

    def radial2(cd):
        sq = (cd * cd).reshape(R, 2 * D)
        err = sq - sq.astype(jnp.bfloat16).astype(f32)
        return (jnp.dot(sq, p62_ref[:], preferred_element_type=f32)
                + jnp.dot(err, p62_ref[:], preferred_element_type=f32))

    cd0 = cdiff(x2)
    ea2 = radial2(cd0)

    coord2 = x2
    for l in range(L):
        if l == 0:
            cd, rad2 = cd0, ea2
        else:
            cd = cdiff(coord2)
            rad2 = radial2(cd)

        A = (jnp.dot(hn, eW1r_ref[l], preferred_element_type=f32)
             + eb1_ref[l])
        C = jnp.dot(hn, eW1c_ref[l], preferred_element_type=f32)
        me = _sh(eb2_ref[l])
        cs6 = jnp.dot(_sh(jnp.dot(me, cW1_ref[l],
                                  preferred_element_type=f32)
                          + cb1_ref[l]),
                      cW26_ref[l], preferred_element_type=f32)
        m1d = _sh(A + C)
        md = _sh(jnp.dot(m1d, eW2_ref[l], preferred_element_type=f32)
                 + eb2_ref[l])
        A3 = A.reshape(Q, Pp, W2)
        C3 = C.reshape(Q, Pp, W2) + nkill
        Ab = lax.broadcast_in_dim(A3, (Q, Pp, Pp, W2), (0, 1, 3)).reshape(R, W2)
        Cb = lax.broadcast_in_dim(C3, (Q, Pp, Pp, W2), (0, 2, 3)).reshape(R, W2)
        z1 = (Ab + Cb
              + jnp.dot(rad2, Wr2_ref[l], preferred_element_type=f32)
              + jnp.dot(ea2, We2_ref[l], preferred_element_type=f32))
        m1 = _sh(z1)
        m = _sh(jnp.dot(m1, eW2_ref[l], preferred_element_type=f32)
                + eb2_ref[l])
        c1 = _sh(jnp.dot(m, cW1_ref[l], preferred_element_type=f32)
                 + cb1_ref[l])
        cm6 = jnp.dot(c1, cW26_ref[l], preferred_element_type=f32)
        upd = jnp.sum(cd * cm6.reshape(Q, Pp, Pp, 2 * D), axis=2)
        coord2 = (coord2 + upd
                  - 2.0 * cs6.reshape(1, 1, 2 * D) * coord2) * nmask3

        if l < L - 1:
            agg = (jnp.sum(m.reshape(Q, Pp, Pp, W2), axis=2)
                   .reshape(NR, W2)) - md - 2.0 * me
            zn = (jnp.dot(hn, nW1h_ref[l], preferred_element_type=f32)
                  + jnp.dot(agg, nW1a_ref[l], preferred_element_type=f32)
                  + nb1_ref[l])
            hn = hn + jnp.dot(_sh(zn), nW2_ref[l],
                              preferred_element_type=f32) + nb2_ref[l]

    vel2 = coord2 - x2
    mean = jnp.sum(vel2 * nmask3, axis=1, keepdims=True) * (1.0 / P)
    o_ref[:] = vel2 - mean


def _blkdiag(Wl):
    z = jnp.zeros_like(Wl)
    top = jnp.concatenate([Wl, z], axis=-1)
    bot = jnp.concatenate([z, Wl], axis=-1)
    return jnp.concatenate([top, bot], axis=1)


def _dup(bl):
    return jnp.concatenate([bl, bl], axis=-1)


def kernel(t, xs, h_initial, edges, emb_W, emb_b, out_W, out_b,
           eW1, eb1, eW2, eb2, nW1, nb1, nW2, nb2, cW1, cb1, cW2):
    del edges, out_W, out_b
    f32 = jnp.float32
    half = B // 2
    x = xs.reshape(B, P, D)
    xpad = jnp.pad(x, ((0, 0), (0, Pp - P), (0, 0)))
    xpack = (xpad.reshape(half, 2, Pp, D).transpose(0, 2, 1, 3)
             .reshape(half, Pp, 2 * D))
    tpack = t.reshape(half, 2)
    hi_pad = jnp.pad(h_initial, ((0, Pp - P), (0, 0)))

    embWh = emb_W[:4]
    embWt = emb_W[4:5].astype(jnp.bfloat16).astype(f32)
    embb = emb_b.reshape(1, H)
    eW1r = _blkdiag(0.5 * eW1[:, :H, :])
    eW1c = _blkdiag(0.5 * eW1[:, H:2 * H, :])
    Wr2 = _blkdiag(0.5 * eW1[:, 2 * H:2 * H + 1, :])
    We2 = _blkdiag(0.5 * eW1[:, 2 * H + 1:, :])
    eb1r = _dup(0.5 * eb1.reshape(L, 1, H))
    eW2d = _blkdiag(0.5 * eW2)
    eb2d = _dup(0.5 * eb2.reshape(L, 1, H))
    cW1d = _blkdiag(0.5 * cW1)
    cb1d = _dup(0.5 * cb1.reshape(L, 1, H))
    cw2rep = jnp.tile(cW2, (1, 1, 3))
    zw = jnp.zeros_like(cw2rep)
    cW26d = jnp.concatenate([jnp.concatenate([cw2rep, zw], -1),
                             jnp.concatenate([zw, cw2rep], -1)], 1)
    nW1h = _blkdiag(0.5 * nW1[:, :H, :])
    nW1a = _blkdiag(0.5 * nW1[:, H:, :])
    nb1r = _dup(0.5 * nb1.reshape(L, 1, H))
    nW2d = _blkdiag(nW2)
    nb2r = _dup(nb2.reshape(L, 1, H))
    p62 = jnp.asarray(_PAT62)
    p2h = jnp.asarray(_PAT2H)

    G = half // Q
    full = lambda *shape: pl.BlockSpec(shape, lambda g: (0,) * len(shape))
    out = pl.pallas_call(
        _egnn_block,
        grid=(G,),
        in_specs=[
            pl.BlockSpec((Q, 2), lambda g: (g, 0)),
            pl.BlockSpec((Q, Pp, 2 * D), lambda g: (g, 0, 0)),
            full(Pp, 4),
            full(4, H), full(1, H), full(1, H),
            full(2 * D, 2), full(2, W2),
            full(L, W2, W2), full(L, W2, W2),
            full(L, 2, W2), full(L, 2, W2), full(L, 1, W2),
            full(L, W2, W2), full(L, 1, W2),
            full(L, W2, W2), full(L, 1, W2), full(L, W2, 2 * D),
            full(L, W2, W2), full(L, W2, W2), full(L, 1, W2),
            full(L, W2, W2), full(L, 1, W2),
        ],
        out_specs=pl.BlockSpec((Q, Pp, 2 * D), lambda g: (g, 0, 0)),
        out_shape=jax.ShapeDtypeStruct((half, Pp, 2 * D), f32),
        compiler_params=pltpu.CompilerParams(
            dimension_semantics=("parallel",)),
    )(tpack, xpack, hi_pad, embWh, embWt, embb, p62, p2h,
      eW1r, eW1c, Wr2, We2, eb1r, eW2d, eb2d, cW1d, cb1d, cW26d,
      nW1h, nW1a, nb1r, nW2d, nb2r)
    vel = (out.reshape(half, Pp, 2, D).transpose(0, 2, 1, 3)
           .reshape(B, Pp, D))
    return vel[:, :P, :].reshape(B, P * D)
